# Initial kernel scaffold; baseline (speedup 1.0000x reference)
import jax
import jax.numpy as jnp
from jax.experimental import pallas as pl


def kernel(user_emb_table, food_emb_table, edge_index):
    raise NotImplementedError("write your pallas kernel here")


# speedup vs baseline: 6.5523x; 6.5523x over previous
import functools

import jax
import jax.numpy as jnp
from jax import lax
from jax.experimental import pallas as pl
from jax.experimental.pallas import tpu as pltpu
from jax.experimental.pallas import tpu_sc as plsc

NU = 100000
H = 128
E = 600000
NL = 3

NS = 16
L = 16

C = 4096
NCH = (NU + C - 1) // C
DIS_PAD = NCH * C
RPT = C // NS
WB = 32

EPAD = E + 128

DEG_TROWS = 296
DEG_ROWS = DEG_TROWS * NS
DEG_N = 100096
DEG_TSLICE = DEG_N // NS

F32 = jnp.float32
I32 = jnp.int32


def _deg_body(uidx, fidx, degu, degf, dtab, ib2, ones, zb):
    c = lax.axis_index("c")
    s = lax.axis_index("s")

    def fill_z(i, _):
        zb[pl.ds(i * L, L)] = jnp.zeros((L,), F32)
        return 0
    lax.fori_loop(0, DEG_TSLICE // L, fill_z, 0)
    for g in range(128 // L):
        ones[pl.ds(g * L, L)] = jnp.ones((L,), F32)
    pltpu.sync_copy(zb, dtab.at[pl.ds(s * DEG_TSLICE, DEG_TSLICE)])
    plsc.subcore_barrier()

    def _histo(idx2, out):
        def macro(m, _):
            row = s * DEG_TROWS + m * 8
            pltpu.sync_copy(idx2.at[pl.ds(row, 8), :], ib2)
            for j in range(8):
                pltpu.sync_copy(ones, dtab.at[ib2.at[j]], add=True)
            return 0
        lax.fori_loop(0, DEG_TROWS // 8, macro, 0)
        plsc.subcore_barrier()
        pltpu.sync_copy(dtab.at[pl.ds(s * DEG_TSLICE, DEG_TSLICE)], zb)
        pltpu.sync_copy(zb, out.at[pl.ds(s * DEG_TSLICE, DEG_TSLICE)])

    @pl.when(c == 0)
    def _():
        _histo(uidx, degu)

    @pl.when(c == 1)
    def _():
        _histo(fidx, degf)


def _spmm_body(tbl, sidx, dloc, cs, dis, f_out, b_out,
               acc, cs_v, si0, di0, rb0, db, wb, wb2, zb, sem):
    c = lax.axis_index("c")
    s = lax.axis_index("s")
    iota16 = jnp.arange(L, dtype=I32)

    def fill_z(i, _):
        r = i // 8
        g = i % 8
        zb[r, pl.ds(g * L, L)] = jnp.zeros((L,), F32)
        return 0
    lax.fori_loop(0, WB * 8, fill_z, 0)

    def zset(k, _):
        pltpu.sync_copy(zb, acc.at[pl.ds(s * RPT + k * WB, WB), :])
        return 0
    lax.fori_loop(0, RPT // WB, zset, 0)
    pltpu.sync_copy(cs.at[pl.ds(0, 48)], cs_v)
    plsc.subcore_barrier()

    nch_c = (NCH - c + 1) // 2

    def chunk_body(i, _):
        ch = c + 2 * i
        csv = cs_v[pl.ds(ch, L)]
        lo = csv[0]
        hi = csv[1]
        cnt = hi - lo
        mylo = lo + (s * cnt) // NS
        myhi = lo + ((s + 1) * cnt) // NS
        ab = (mylo // 8) * 8
        nb = (myhi - ab + 127) // 128

        def batch_body(j, _):
            base = ab + j * 128
            pltpu.sync_copy(sidx.at[pl.ds(base, 128)], si0)
            pltpu.sync_copy(dloc.at[pl.ds(base, 128)], di0)
            for g in range(8):
                pos = base + g * L + iota16
                v = di0[pl.ds(g * L, L)]
                m = (pos >= mylo) & (pos < myhi)
                di0[pl.ds(g * L, L)] = jnp.where(m, v, C)
            pltpu.async_copy(tbl.at[si0], rb0, sem).wait()
            pltpu.sync_copy(rb0, acc.at[di0], add=True)
            return 0
        lax.fori_loop(0, nb, batch_body, 0)
        plsc.subcore_barrier()

        row0 = ch * C + s * RPT
        pltpu.sync_copy(dis.at[pl.ds(row0, RPT)], db.at[pl.ds(0, RPT)])

        def wb_body(k, _):
            r0 = row0 + k * WB

            @pl.when(r0 < NU)
            def _():
                off = s * RPT + k * WB
                pltpu.sync_copy(acc.at[pl.ds(off, WB), :], wb)

                def row_body(r, _):
                    dv = db[pl.ds(k * WB + r, L)]
                    dvb = jnp.broadcast_to(dv[0], (L,))
                    for g in range(8):
                        x = wb[r, pl.ds(g * L, L)] * dvb
                        wb[r, pl.ds(g * L, L)] = x
                        wb2[r, pl.ds(g * L, L)] = x * dvb
                    return 0
                lax.fori_loop(0, WB, row_body, 0)
                pltpu.sync_copy(wb, f_out.at[pl.ds(r0, WB), :])
                pltpu.sync_copy(wb2, b_out.at[pl.ds(r0, WB), :])
                pltpu.sync_copy(zb, acc.at[pl.ds(off, WB), :])
            return 0
        lax.fori_loop(0, RPT // WB, wb_body, 0)
        plsc.subcore_barrier()
        return 0

    lax.fori_loop(0, nch_c, chunk_body, 0)


def _rsqrt_tc(x_ref, o_ref):
    o_ref[...] = lax.rsqrt(x_ref[...] + 1e-8)


def _scale_tc(x_ref, d_ref, o_ref):
    o_ref[...] = x_ref[...] * d_ref[...]


def _mean_tc(a_ref, b_ref, c_ref, d_ref, o_ref):
    o_ref[...] = (a_ref[...] + b_ref[...] + c_ref[...] + d_ref[...]) * 0.25


def _row_block(i):
    return (i, 0)


def kernel(user_emb_table, food_emb_table, edge_index):
    return _pipeline(user_emb_table, food_emb_table, edge_index, NL)


def _pipeline(user_emb_table, food_emb_table, edge_index, n_layers):
    ui = edge_index[0].astype(I32)
    fi = edge_index[1].astype(I32)

    upad = jnp.pad(ui, (0, DEG_ROWS * 128 - E),
                   constant_values=NU).reshape(DEG_ROWS, 128)
    fpad = jnp.pad(fi, (0, DEG_ROWS * 128 - E),
                   constant_values=NU).reshape(DEG_ROWS, 128)

    koff = jnp.int32(1 << 17)
    keys = jnp.concatenate([fi, ui + koff])
    vals = jnp.concatenate([ui, fi])
    ks, vs = lax.sort_key_val(keys, vals)
    d_f, s_f = ks[:E], vs[:E]
    d_u, s_u = ks[E:] - koff, vs[E:]
    sf_p = jnp.pad(s_f, (0, EPAD - E))
    su_p = jnp.pad(s_u, (0, EPAD - E))
    dlf_p = jnp.pad(d_f & (C - 1), (0, EPAD - E), constant_values=C)
    dlu_p = jnp.pad(d_u & (C - 1), (0, EPAD - E), constant_values=C)

    mesh = plsc.VectorSubcoreMesh(core_axis_name="c", subcore_axis_name="s")
    deg_k = pl.kernel(
        _deg_body,
        out_type=(jax.ShapeDtypeStruct((DEG_N,), F32),
                  jax.ShapeDtypeStruct((DEG_N,), F32)),
        mesh=mesh,
        scratch_types=[
            pltpu.VMEM_SHARED((DEG_N,), F32),
            pltpu.VMEM((8, 128), I32),
            pltpu.VMEM((128,), F32),
            pltpu.VMEM((DEG_TSLICE,), F32),
        ],
    )
    degu, degf = deg_k(upad, fpad)

    def _chunk_starts(deg):
        per_chunk = jnp.sum(
            jnp.pad(deg[:NU].astype(I32), (0, DIS_PAD - NU)).reshape(NCH, C),
            axis=1)
        cs = jnp.concatenate([jnp.zeros((1,), I32), jnp.cumsum(per_chunk)])
        return jnp.pad(cs, (0, 48 - NCH - 1))

    cs_u = _chunk_starts(degu)
    cs_f = _chunk_starts(degf)

    rsq = pl.pallas_call(
        _rsqrt_tc,
        out_shape=jax.ShapeDtypeStruct((DEG_N // 128, 128), F32),
    )
    disu = rsq(degu.reshape(DEG_N // 128, 128)).reshape(DEG_N)[:NU]
    disf = rsq(degf.reshape(DEG_N // 128, 128)).reshape(DEG_N)[:NU]
    disu_p = jnp.pad(disu, (0, DIS_PAD - NU))
    disf_p = jnp.pad(disf, (0, DIS_PAD - NU))

    scale = pl.pallas_call(
        _scale_tc,
        grid=(20,),
        in_specs=[pl.BlockSpec((5000, 128), _row_block),
                  pl.BlockSpec((5000, 1), _row_block)],
        out_specs=pl.BlockSpec((5000, 128), _row_block),
        out_shape=jax.ShapeDtypeStruct((NU, H), F32),
    )
    a = scale(user_emb_table, disu.reshape(NU, 1))
    b = scale(food_emb_table, disf.reshape(NU, 1))

    spmm = pl.kernel(
        _spmm_body,
        out_type=(jax.ShapeDtypeStruct((NU, H), F32),
                  jax.ShapeDtypeStruct((NU, H), F32)),
        mesh=mesh,
        scratch_types=[
            pltpu.VMEM_SHARED((C + 8, H), F32),
            pltpu.VMEM((48,), I32),
            pltpu.VMEM((128,), I32),
            pltpu.VMEM((128,), I32),
            pltpu.VMEM((128, H), F32),
            pltpu.VMEM((RPT + L,), F32),
            pltpu.VMEM((WB, H), F32),
            pltpu.VMEM((WB, H), F32),
            pltpu.VMEM((WB, H), F32),
            pltpu.SemaphoreType.DMA,
        ],
    )

    u_list = [user_emb_table]
    f_list = [food_emb_table]
    for _ in range(n_layers):
        f_next, b_next = spmm(a, sf_p, dlf_p, cs_f, disf_p)
        u_next, a_next = spmm(b, su_p, dlu_p, cs_u, disu_p)
        a, b = a_next, b_next
        u_list.append(u_next)
        f_list.append(f_next)

    if n_layers < NL:
        return (u_list[-1], f_list[-1])

    mean = pl.pallas_call(
        _mean_tc,
        grid=(20,),
        in_specs=[pl.BlockSpec((5000, 128), _row_block)] * 4,
        out_specs=pl.BlockSpec((5000, 128), _row_block),
        out_shape=jax.ShapeDtypeStruct((NU, H), F32),
    )
    return (mean(*u_list), mean(*f_list))


# speedup vs baseline: 8.9012x; 1.3585x over previous
import functools

import jax
import jax.numpy as jnp
from jax import lax
from jax.experimental import pallas as pl
from jax.experimental.pallas import tpu as pltpu
from jax.experimental.pallas import tpu_sc as plsc

NU = 100000
H = 128
E = 600000
NL = 3

NS = 16
L = 16

C = 3584
NCH = (NU + C - 1) // C
DIS_PAD = NCH * C
RPT = C // NS
WB = 32

EPAD = E + 384

DEG_TROWS = 296
DEG_ROWS = DEG_TROWS * NS
DEG_N = 100096
DEG_TSLICE = DEG_N // NS

F32 = jnp.float32
I32 = jnp.int32


def _deg_body(uidx, fidx, degu, degf, dtab, ib2, ones, zb):
    c = lax.axis_index("c")
    s = lax.axis_index("s")

    def fill_z(i, _):
        zb[pl.ds(i * L, L)] = jnp.zeros((L,), F32)
        return 0
    lax.fori_loop(0, DEG_TSLICE // L, fill_z, 0)
    for g in range(128 // L):
        ones[pl.ds(g * L, L)] = jnp.ones((L,), F32)
    pltpu.sync_copy(zb, dtab.at[pl.ds(s * DEG_TSLICE, DEG_TSLICE)])
    plsc.subcore_barrier()

    def _histo(idx2, out):
        def macro(m, _):
            row = s * DEG_TROWS + m * 8
            pltpu.sync_copy(idx2.at[pl.ds(row, 8), :], ib2)
            for j in range(8):
                pltpu.sync_copy(ones, dtab.at[ib2.at[j]], add=True)
            return 0
        lax.fori_loop(0, DEG_TROWS // 8, macro, 0)
        plsc.subcore_barrier()
        pltpu.sync_copy(dtab.at[pl.ds(s * DEG_TSLICE, DEG_TSLICE)], zb)
        pltpu.sync_copy(zb, out.at[pl.ds(s * DEG_TSLICE, DEG_TSLICE)])

    @pl.when(c == 0)
    def _():
        _histo(uidx, degu)

    @pl.when(c == 1)
    def _():
        _histo(fidx, degf)


def _spmm_body(tbl, sd, cs, dis, f_out, b_out,
               acc, cs_v, sd0, sd1, rb0, rb1, db, wb, wb2, zb,
               sg0, sg1, ss0, ss1, swf, swb):
    c = lax.axis_index("c")
    s = lax.axis_index("s")
    iota16 = jnp.arange(L, dtype=I32)

    def wait_gather(sem, rb):
        pltpu.make_async_copy(tbl.at[pl.ds(0, 128), :], rb, sem).wait()

    def wait_scatter(sem, rb):
        pltpu.make_async_copy(tbl.at[pl.ds(0, 128), :], rb, sem).wait()

    def wait_wb(sem, w):
        pltpu.make_async_copy(f_out.at[pl.ds(0, RPT), :], w, sem).wait()

    def fill_z(i, _):
        r = i // 8
        g = i % 8
        zb[r, pl.ds(g * L, L)] = jnp.zeros((L,), F32)
        return 0
    lax.fori_loop(0, WB * 8, fill_z, 0)

    def zset(k, _):
        pltpu.sync_copy(zb, acc.at[pl.ds(s * RPT + k * WB, WB), :])
        return 0
    lax.fori_loop(0, RPT // WB, zset, 0)
    pltpu.sync_copy(cs.at[pl.ds(0, 48)], cs_v)
    plsc.subcore_barrier()

    nch_c = (NCH - c + 1) // 2

    def chunk_body(i, wflag):
        ch = c + 2 * i
        csv = cs_v[pl.ds(ch, L)]
        lo = csv[0]
        hi = csv[1]
        cnt = hi - lo
        mylo = lo + (s * cnt) // NS
        myhi = lo + ((s + 1) * cnt) // NS
        ab = (mylo // 128) * 128
        nb2 = (myhi - ab + 255) // 256
        nb = nb2 * 2

        def load_mask(jb, sdv):
            base = pl.multiple_of(ab + jb * 128, 128)
            pltpu.sync_copy(sd.at[:, pl.ds(base, 128)], sdv)
            interior = (base >= mylo) & (base + 128 <= myhi)

            @pl.when(jnp.logical_not(interior))
            def _():
                for g in range(8):
                    pos = base + g * L + iota16
                    v = sdv[1, pl.ds(g * L, L)]
                    m = (pos >= mylo) & (pos < myhi)
                    sdv[1, pl.ds(g * L, L)] = jnp.where(m, v, C)

        @pl.when(nb > 0)
        def _():
            load_mask(0, sd0)
            pltpu.async_copy(tbl.at[sd0.at[0]], rb0, sg0)

        def pair(j2, carry):
            j = 2 * j2

            @pl.when(j2 > 0)
            def _():
                wait_scatter(ss1, rb1)
            load_mask(j + 1, sd1)
            pltpu.async_copy(tbl.at[sd1.at[0]], rb1, sg1)
            wait_gather(sg0, rb0)
            pltpu.async_copy(rb0, acc.at[sd0.at[1]], ss0, add=True)

            @pl.when(j + 2 < nb)
            def _():
                wait_scatter(ss0, rb0)
                load_mask(j + 2, sd0)
                pltpu.async_copy(tbl.at[sd0.at[0]], rb0, sg0)
            wait_gather(sg1, rb1)
            pltpu.async_copy(rb1, acc.at[sd1.at[1]], ss1, add=True)
            return carry
        lax.fori_loop(0, nb2, pair, 0)

        @pl.when(nb > 0)
        def _():
            wait_scatter(ss0, rb0)
            wait_scatter(ss1, rb1)
        plsc.subcore_barrier()

        row0 = ch * C + s * RPT

        @pl.when(wflag == 1)
        def _():
            wait_wb(swf, wb)
            wait_wb(swb, wb2)
        pltpu.sync_copy(dis.at[pl.ds(row0, RPT)], db.at[pl.ds(0, RPT)])
        pltpu.sync_copy(acc.at[pl.ds(s * RPT, RPT), :], wb)

        def zset2(k, _):
            pltpu.sync_copy(zb, acc.at[pl.ds(s * RPT + k * WB, WB), :])
            return 0
        lax.fori_loop(0, RPT // WB, zset2, 0)

        def row_body(r, _):
            dv = db[pl.ds(r, L)]
            dvb = jnp.broadcast_to(dv[0], (L,))
            for g in range(8):
                x = wb[r, pl.ds(g * L, L)] * dvb
                wb[r, pl.ds(g * L, L)] = x
                wb2[r, pl.ds(g * L, L)] = x * dvb
            return 0
        lax.fori_loop(0, RPT, row_body, 0)

        full = (row0 + RPT) <= NU

        @pl.when(full)
        def _():
            pltpu.async_copy(wb, f_out.at[pl.ds(row0, RPT), :], swf)
            pltpu.async_copy(wb2, b_out.at[pl.ds(row0, RPT), :], swb)

        @pl.when(jnp.logical_not(full))
        def _():
            def sub(k, _):
                r0 = row0 + k * WB

                @pl.when(r0 < NU)
                def _():
                    pltpu.sync_copy(wb.at[pl.ds(k * WB, WB), :],
                                    f_out.at[pl.ds(r0, WB), :])
                    pltpu.sync_copy(wb2.at[pl.ds(k * WB, WB), :],
                                    b_out.at[pl.ds(r0, WB), :])
                return 0
            lax.fori_loop(0, RPT // WB, sub, 0)
        plsc.subcore_barrier()
        return jnp.where(full, jnp.int32(1), jnp.int32(0))

    wflag = lax.fori_loop(0, nch_c, chunk_body, jnp.int32(0))

    @pl.when(wflag == 1)
    def _():
        wait_wb(swf, wb)
        wait_wb(swb, wb2)


def _make_spmm(mesh):
    return pl.kernel(
        _spmm_body,
        out_type=(jax.ShapeDtypeStruct((NU, H), F32),
                  jax.ShapeDtypeStruct((NU, H), F32)),
        mesh=mesh,
        scratch_types=[
            pltpu.VMEM_SHARED((C + 8, H), F32),
            pltpu.VMEM((48,), I32),
            pltpu.VMEM((2, 128), I32),
            pltpu.VMEM((2, 128), I32),
            pltpu.VMEM((128, H), F32),
            pltpu.VMEM((128, H), F32),
            pltpu.VMEM((RPT + L,), F32),
            pltpu.VMEM((RPT, H), F32),
            pltpu.VMEM((RPT, H), F32),
            pltpu.VMEM((WB, H), F32),
            pltpu.SemaphoreType.DMA,
            pltpu.SemaphoreType.DMA,
            pltpu.SemaphoreType.DMA,
            pltpu.SemaphoreType.DMA,
            pltpu.SemaphoreType.DMA,
            pltpu.SemaphoreType.DMA,
        ],
    )


def _rsqrt_tc(x_ref, o_ref):
    o_ref[...] = lax.rsqrt(x_ref[...] + 1e-8)


def _scale_tc(x_ref, d_ref, o_ref):
    o_ref[...] = x_ref[...] * d_ref[...]


def _mean_tc(a_ref, b_ref, c_ref, d_ref, o_ref):
    o_ref[...] = (a_ref[...] + b_ref[...] + c_ref[...] + d_ref[...]) * 0.25


def _row_block(i):
    return (i, 0)


def kernel(user_emb_table, food_emb_table, edge_index):
    return _pipeline(user_emb_table, food_emb_table, edge_index, NL)


def _pipeline(user_emb_table, food_emb_table, edge_index, n_layers):
    ui = edge_index[0].astype(I32)
    fi = edge_index[1].astype(I32)

    upad = jnp.pad(ui, (0, DEG_ROWS * 128 - E),
                   constant_values=NU).reshape(DEG_ROWS, 128)
    fpad = jnp.pad(fi, (0, DEG_ROWS * 128 - E),
                   constant_values=NU).reshape(DEG_ROWS, 128)

    koff = jnp.int32(1 << 17)
    keys = jnp.concatenate([fi, ui + koff])
    vals = jnp.concatenate([ui, fi])
    ks, vs = lax.sort_key_val(keys, vals)
    d_f, s_f = ks[:E], vs[:E]
    d_u, s_u = ks[E:] - koff, vs[E:]
    sd_f = jnp.stack([jnp.pad(s_f, (0, EPAD - E)),
                      jnp.pad(d_f % C, (0, EPAD - E), constant_values=C)])
    sd_u = jnp.stack([jnp.pad(s_u, (0, EPAD - E)),
                      jnp.pad(d_u % C, (0, EPAD - E), constant_values=C)])

    mesh = plsc.VectorSubcoreMesh(core_axis_name="c", subcore_axis_name="s")
    deg_k = pl.kernel(
        _deg_body,
        out_type=(jax.ShapeDtypeStruct((DEG_N,), F32),
                  jax.ShapeDtypeStruct((DEG_N,), F32)),
        mesh=mesh,
        scratch_types=[
            pltpu.VMEM_SHARED((DEG_N,), F32),
            pltpu.VMEM((8, 128), I32),
            pltpu.VMEM((128,), F32),
            pltpu.VMEM((DEG_TSLICE,), F32),
        ],
    )
    degu, degf = deg_k(upad, fpad)

    def _chunk_starts(deg):
        per_chunk = jnp.sum(
            jnp.pad(deg[:NU].astype(I32), (0, DIS_PAD - NU)).reshape(NCH, C),
            axis=1)
        cs = jnp.concatenate([jnp.zeros((1,), I32), jnp.cumsum(per_chunk)])
        return jnp.pad(cs, (0, 48 - NCH - 1))

    cs_u = _chunk_starts(degu)
    cs_f = _chunk_starts(degf)

    rsq = pl.pallas_call(
        _rsqrt_tc,
        out_shape=jax.ShapeDtypeStruct((DEG_N // 128, 128), F32),
    )
    disu = rsq(degu.reshape(DEG_N // 128, 128)).reshape(DEG_N)[:NU]
    disf = rsq(degf.reshape(DEG_N // 128, 128)).reshape(DEG_N)[:NU]
    disu_p = jnp.pad(disu, (0, DIS_PAD - NU))
    disf_p = jnp.pad(disf, (0, DIS_PAD - NU))

    scale = pl.pallas_call(
        _scale_tc,
        grid=(20,),
        in_specs=[pl.BlockSpec((5000, 128), _row_block),
                  pl.BlockSpec((5000, 1), _row_block)],
        out_specs=pl.BlockSpec((5000, 128), _row_block),
        out_shape=jax.ShapeDtypeStruct((NU, H), F32),
    )
    a = scale(user_emb_table, disu.reshape(NU, 1))
    b = scale(food_emb_table, disf.reshape(NU, 1))

    spmm = _make_spmm(mesh)

    u_list = [user_emb_table]
    f_list = [food_emb_table]
    for _ in range(n_layers):
        f_next, b_next = spmm(a, sd_f, cs_f, disf_p)
        u_next, a_next = spmm(b, sd_u, cs_u, disu_p)
        a, b = a_next, b_next
        u_list.append(u_next)
        f_list.append(f_next)

    if n_layers < NL:
        return (u_list[-1], f_list[-1])

    mean = pl.pallas_call(
        _mean_tc,
        grid=(20,),
        in_specs=[pl.BlockSpec((5000, 128), _row_block)] * 4,
        out_specs=pl.BlockSpec((5000, 128), _row_block),
        out_shape=jax.ShapeDtypeStruct((NU, H), F32),
    )
    return (mean(*u_list), mean(*f_list))
